# baseline (speedup 1.0000x reference)
import functools

import jax
import jax.numpy as jnp
import numpy as np
from jax.experimental import pallas as pl
from jax.experimental.pallas import tpu as pltpu

_ROWS_PER_BLOCK = 32
_COLS_PER_BLOCK = 16384

_LN2 = np.float32(0.6931472)
_LANES = 128

_TINY = np.float32(np.finfo(np.float32).tiny)
_NEG_INF = np.float32(-np.inf)
_INT_MAX = np.int32(0x7FFFFFFF)

_KS0 = 0
_KS1 = 42
_KS2 = 0x1BD11BDA ^ _KS0 ^ _KS1
_ROT0 = (13, 15, 26, 6)
_ROT1 = (17, 29, 16, 24)


def _rotl(x, r):
    return (x << jnp.uint32(r)) | (x >> jnp.uint32(32 - r))


def _threefry_bits(lin):
    x1 = lin + jnp.uint32(_KS1)
    x0 = x1
    x1 = _rotl(x1, _ROT0[0]) ^ x0
    for r in _ROT0[1:]:
        x0 = x0 + x1
        x1 = _rotl(x1, r) ^ x0
    x0 = x0 + jnp.uint32(_KS1)
    x1 = x1 + jnp.uint32((_KS2 + 1) & 0xFFFFFFFF)
    for r in _ROT1:
        x0 = x0 + x1
        x1 = _rotl(x1, r) ^ x0
    x0 = x0 + jnp.uint32(_KS2)
    x1 = x1 + jnp.uint32(_KS0 + 2)
    for r in _ROT0:
        x0 = x0 + x1
        x1 = _rotl(x1, r) ^ x0
    x1 = x1 + jnp.uint32(_KS1 + 3)
    for r in _ROT1:
        x0 = x0 + x1
        x1 = _rotl(x1, r) ^ x0
    x0 = x0 + jnp.uint32(_KS1)
    x1 = x1 + jnp.uint32((_KS2 + 4) & 0xFFFFFFFF)
    for r in _ROT0:
        x0 = x0 + x1
        x1 = _rotl(x1, r) ^ x0
    x0 = x0 + jnp.uint32(_KS2)
    x1 = x1 + jnp.uint32(_KS0 + 5)
    return x0 ^ x1


def _sample_kernel(logits_ref, out_ref, max_ref, arg_ref, *, n_cols, n_col_blocks):
    r = pl.program_id(0)
    c = pl.program_id(1)
    rows = _ROWS_PER_BLOCK

    @pl.when(c == 0)
    def _init():
        max_ref[...] = jnp.full_like(max_ref, _NEG_INF)
        arg_ref[...] = jnp.full_like(arg_ref, _INT_MAX)

    shape = (rows, _LANES)
    lane = jax.lax.broadcasted_iota(jnp.int32, shape, 1)
    row = r * rows + jax.lax.broadcasted_iota(jnp.int32, shape, 0)
    row_base = row * n_cols
    lin0 = row_base + c * _COLS_PER_BLOCK + lane
    bound = row_base + n_cols

    run_max = max_ref[...]
    run_arg = arg_ref[...]
    for j in range(_COLS_PER_BLOCK // _LANES):
        lin = lin0 + j * _LANES
        bits = _threefry_bits(lin.astype(jnp.uint32))
        fb = (bits >> jnp.uint32(9)) | jnp.uint32(0x3F800000)
        u = jax.lax.bitcast_convert_type(fb, jnp.float32) - jnp.float32(1.0)
        u = jnp.maximum(u, _TINY)
        e = -jnp.log(u)
        val = logits_ref[:, j * _LANES : (j + 1) * _LANES] - jnp.log(e)
        val = jnp.where(lin < bound, val, _NEG_INF)
        upd = val > run_max
        run_max = jnp.where(upd, val, run_max)
        run_arg = jnp.where(upd, lin, run_arg)
    max_ref[...] = run_max
    arg_ref[...] = run_arg

    @pl.when(c == n_col_blocks - 1)
    def _finalize():
        m = max_ref[...]
        row_max = jnp.max(m, axis=1, keepdims=True)
        ids = jnp.where(m == row_max, arg_ref[...], _INT_MAX)
        out_ref[...] = jnp.min(ids, axis=1, keepdims=True) - row_base[:, :1]


def kernel(logits):
    n_rows, n_cols = logits.shape
    n_row_blocks = pl.cdiv(n_rows, _ROWS_PER_BLOCK)
    n_col_blocks = pl.cdiv(n_cols, _COLS_PER_BLOCK)

    out = pl.pallas_call(
        functools.partial(
            _sample_kernel, n_cols=n_cols, n_col_blocks=n_col_blocks
        ),
        grid=(n_row_blocks, n_col_blocks),
        in_specs=[
            pl.BlockSpec((_ROWS_PER_BLOCK, _COLS_PER_BLOCK), lambda r, c: (r, c)),
        ],
        out_specs=pl.BlockSpec((_ROWS_PER_BLOCK, 1), lambda r, c: (r, 0)),
        out_shape=jax.ShapeDtypeStruct((n_rows, 1), jnp.int32),
        scratch_shapes=[
            pltpu.VMEM((_ROWS_PER_BLOCK, _LANES), jnp.float32),
            pltpu.VMEM((_ROWS_PER_BLOCK, _LANES), jnp.int32),
        ],
    )(logits)
    return out[:, 0]


# speedup vs baseline: 1.0303x; 1.0303x over previous
import functools

import jax
import jax.numpy as jnp
import numpy as np
from jax.experimental import pallas as pl
from jax.experimental.pallas import tpu as pltpu
from jax.sharding import Mesh, PartitionSpec as P

_ROWS_PER_BLOCK = 32
_COLS_PER_BLOCK = 16384
_LANES = 128

_TINY = np.float32(np.finfo(np.float32).tiny)
_NEG_INF = np.float32(-np.inf)
_INT_MAX = np.int32(0x7FFFFFFF)

_KS0 = 0
_KS1 = 42
_KS2 = 0x1BD11BDA ^ _KS0 ^ _KS1
_ROT0 = (13, 15, 26, 6)
_ROT1 = (17, 29, 16, 24)


def _rotl(x, r):
    return (x << jnp.uint32(r)) | (x >> jnp.uint32(32 - r))


def _threefry_bits(x1):
    x0 = x1
    x1 = _rotl(x1, _ROT0[0]) ^ x0
    for r in _ROT0[1:]:
        x0 = x0 + x1
        x1 = _rotl(x1, r) ^ x0
    x0 = x0 + jnp.uint32(_KS1)
    x1 = x1 + jnp.uint32((_KS2 + 1) & 0xFFFFFFFF)
    for r in _ROT1:
        x0 = x0 + x1
        x1 = _rotl(x1, r) ^ x0
    x0 = x0 + jnp.uint32(_KS2)
    x1 = x1 + jnp.uint32(_KS0 + 2)
    for r in _ROT0:
        x0 = x0 + x1
        x1 = _rotl(x1, r) ^ x0
    x1 = x1 + jnp.uint32(_KS1 + 3)
    for r in _ROT1:
        x0 = x0 + x1
        x1 = _rotl(x1, r) ^ x0
    x0 = x0 + jnp.uint32(_KS1)
    x1 = x1 + jnp.uint32((_KS2 + 4) & 0xFFFFFFFF)
    for r in _ROT0:
        x0 = x0 + x1
        x1 = _rotl(x1, r) ^ x0
    x0 = x0 + jnp.uint32(_KS2)
    x1 = x1 + jnp.uint32(_KS0 + 5)
    return x0 ^ x1


def _sample_kernel(
    off_ref,
    logits_ref,
    outmax_ref,
    outlin_ref,
    max_ref,
    arg_ref,
    *,
    n_total,
    n_local,
    n_col_blocks,
):
    r = pl.program_id(0)
    c = pl.program_id(1)
    rows = _ROWS_PER_BLOCK

    @pl.when(c == 0)
    def _init():
        max_ref[...] = jnp.full_like(max_ref, _NEG_INF)
        arg_ref[...] = jnp.full_like(arg_ref, _INT_MAX)

    shape = (rows, _LANES)
    lane = jax.lax.broadcasted_iota(jnp.int32, shape, 1)
    row = r * rows + jax.lax.broadcasted_iota(jnp.int32, shape, 0)
    row_base = row * n_total
    linp0 = row_base + (off_ref[0] + (c * _COLS_PER_BLOCK + _KS1)) + lane
    bound = row_base + (off_ref[0] + (n_local + _KS1))

    run_max = max_ref[...]
    run_arg = arg_ref[...]
    for j in range(_COLS_PER_BLOCK // _LANES):
        linp = linp0 + j * _LANES
        bits = _threefry_bits(linp.astype(jnp.uint32))
        fb = (bits >> jnp.uint32(9)) | jnp.uint32(0x3F800000)
        u = jax.lax.bitcast_convert_type(fb, jnp.float32) + jnp.float32(-1.0)
        u = jnp.maximum(u, _TINY)
        e = -jnp.log(u)
        val = logits_ref[:, j * _LANES : (j + 1) * _LANES] - jnp.log(e)
        val = jnp.where(linp < bound, val, _NEG_INF)
        upd = val > run_max
        run_max = jnp.where(upd, val, run_max)
        run_arg = jnp.where(upd, linp, run_arg)
    max_ref[...] = run_max
    arg_ref[...] = run_arg

    @pl.when(c == n_col_blocks - 1)
    def _finalize():
        m = max_ref[...]
        row_max = jnp.max(m, axis=1, keepdims=True)
        ids = jnp.where(m == row_max, arg_ref[...], _INT_MAX)
        outmax_ref[...] = row_max
        outlin_ref[...] = jnp.min(ids, axis=1, keepdims=True) - row_base[:, :1]


def _partial_sample(logits, n_total, offset):
    n_rows, n_local = logits.shape
    n_row_blocks = pl.cdiv(n_rows, _ROWS_PER_BLOCK)
    n_col_blocks = pl.cdiv(n_local, _COLS_PER_BLOCK)

    outmax, outlin = pl.pallas_call(
        functools.partial(
            _sample_kernel,
            n_total=n_total,
            n_local=n_local,
            n_col_blocks=n_col_blocks,
        ),
        grid_spec=pltpu.PrefetchScalarGridSpec(
            num_scalar_prefetch=1,
            grid=(n_row_blocks, n_col_blocks),
            in_specs=[
                pl.BlockSpec(
                    (_ROWS_PER_BLOCK, _COLS_PER_BLOCK), lambda r, c, off: (r, c)
                ),
            ],
            out_specs=[
                pl.BlockSpec((_ROWS_PER_BLOCK, 1), lambda r, c, off: (r, 0)),
                pl.BlockSpec((_ROWS_PER_BLOCK, 1), lambda r, c, off: (r, 0)),
            ],
            scratch_shapes=[
                pltpu.VMEM((_ROWS_PER_BLOCK, _LANES), jnp.float32),
                pltpu.VMEM((_ROWS_PER_BLOCK, _LANES), jnp.int32),
            ],
        ),
        out_shape=[
            jax.ShapeDtypeStruct((n_rows, 1), jnp.float32),
            jax.ShapeDtypeStruct((n_rows, 1), jnp.int32),
        ],
    )(jnp.asarray([offset], dtype=jnp.int32), logits)
    return outmax[:, 0], outlin[:, 0]


def kernel(logits):
    n_rows, n_cols = logits.shape
    devs = jax.devices()
    use_shards = len(devs) >= 2 and n_cols % 2 == 0

    if not use_shards:
        _, lin = _partial_sample(logits, n_cols, 0)
        return lin - _KS1

    half = n_cols // 2
    mesh = Mesh(np.asarray(devs[:2]), ("x",))

    def shard_fn(lg):
        off = jax.lax.axis_index("x") * half
        m, lin = _partial_sample(lg, n_cols, off)
        return m[:, None], lin[:, None]

    m2, l2 = jax.shard_map(
        shard_fn,
        mesh=mesh,
        in_specs=P(None, "x"),
        out_specs=P(None, "x"),
        check_vma=False,
    )(logits)
    take1 = m2[:, 1] > m2[:, 0]
    lin = jnp.where(take1, l2[:, 1], l2[:, 0])
    return lin - _KS1


# speedup vs baseline: 1.0341x; 1.0036x over previous
import functools

import jax
import jax.numpy as jnp
import numpy as np
from jax.experimental import pallas as pl
from jax.experimental.pallas import tpu as pltpu
from jax.sharding import Mesh, PartitionSpec as P

_ROWS_PER_BLOCK = 32
_COLS_PER_BLOCK = 16384
_LANES = 128

_TINY = np.float32(np.finfo(np.float32).tiny)
_NEG_INF = np.float32(-np.inf)
_INT_MAX = np.int32(0x7FFFFFFF)

_KS0 = 0
_KS1 = 42
_KS2 = 0x1BD11BDA ^ _KS0 ^ _KS1
_ROT0 = (13, 15, 26, 6)
_ROT1 = (17, 29, 16, 24)


def _rotl(x, r):
    return (x << jnp.uint32(r)) | (x >> jnp.uint32(32 - r))


def _threefry_bits(x1):
    x0 = x1
    x1 = _rotl(x1, _ROT0[0]) ^ x0
    for r in _ROT0[1:]:
        x0 = x0 + x1
        x1 = _rotl(x1, r) ^ x0
    x0 = x0 + jnp.uint32(_KS1)
    x1 = x1 + jnp.uint32((_KS2 + 1) & 0xFFFFFFFF)
    for r in _ROT1:
        x0 = x0 + x1
        x1 = _rotl(x1, r) ^ x0
    x0 = x0 + jnp.uint32(_KS2)
    x1 = x1 + jnp.uint32(_KS0 + 2)
    for r in _ROT0:
        x0 = x0 + x1
        x1 = _rotl(x1, r) ^ x0
    x1 = x1 + jnp.uint32(_KS1 + 3)
    for r in _ROT1:
        x0 = x0 + x1
        x1 = _rotl(x1, r) ^ x0
    x0 = x0 + jnp.uint32(_KS1)
    x1 = x1 + jnp.uint32((_KS2 + 4) & 0xFFFFFFFF)
    for r in _ROT0:
        x0 = x0 + x1
        x1 = _rotl(x1, r) ^ x0
    x0 = x0 + jnp.uint32(_KS2)
    x1 = x1 + jnp.uint32(_KS0 + 5)
    return x0 ^ x1


def _sample_kernel(
    off_ref,
    logits_ref,
    outmax_ref,
    outlin_ref,
    max_ref,
    arg_ref,
    *,
    n_total,
    n_local,
    n_col_blocks,
):
    r = pl.program_id(0)
    c = pl.program_id(1)
    rows = _ROWS_PER_BLOCK

    @pl.when(c == 0)
    def _init():
        max_ref[...] = jnp.full_like(max_ref, _NEG_INF)
        arg_ref[...] = jnp.full_like(arg_ref, _INT_MAX)

    shape = (rows, _LANES)
    lane = jax.lax.broadcasted_iota(jnp.int32, shape, 1)
    row = r * rows + jax.lax.broadcasted_iota(jnp.int32, shape, 0)
    row_base = row * n_total
    linp0 = row_base + (off_ref[0] + (c * _COLS_PER_BLOCK + _KS1)) + lane
    bound = row_base + (off_ref[0] + (n_local + _KS1))

    run_max = max_ref[...]
    run_arg = arg_ref[...]
    for j in range(_COLS_PER_BLOCK // _LANES):
        linp = linp0 + j * _LANES
        bits = _threefry_bits(linp.astype(jnp.uint32))
        fb = (bits >> jnp.uint32(9)) | jnp.uint32(0x3F800000)
        u = jax.lax.bitcast_convert_type(fb, jnp.float32) + jnp.float32(-1.0)
        u = jnp.maximum(u, _TINY)
        e = -jnp.log(u)
        val = logits_ref[:, j * _LANES : (j + 1) * _LANES] - jnp.log(e)
        val = jnp.where(linp < bound, val, _NEG_INF)
        upd = val > run_max
        run_max = jnp.where(upd, val, run_max)
        run_arg = jnp.where(upd, linp, run_arg)
    max_ref[...] = run_max
    arg_ref[...] = run_arg

    @pl.when(c == n_col_blocks - 1)
    def _finalize():
        m = max_ref[...]
        row_max = jnp.max(m, axis=1, keepdims=True)
        ids = jnp.where(m == row_max, arg_ref[...], _INT_MAX)
        outmax_ref[...] = row_max
        outlin_ref[...] = jnp.min(ids, axis=1, keepdims=True) - row_base[:, :1]


def _partial_sample(logits, n_total, offset):
    n_rows, n_local = logits.shape
    n_row_blocks = pl.cdiv(n_rows, _ROWS_PER_BLOCK)
    n_col_blocks = pl.cdiv(n_local, _COLS_PER_BLOCK)

    outmax, outlin = pl.pallas_call(
        functools.partial(
            _sample_kernel,
            n_total=n_total,
            n_local=n_local,
            n_col_blocks=n_col_blocks,
        ),
        grid_spec=pltpu.PrefetchScalarGridSpec(
            num_scalar_prefetch=1,
            grid=(n_row_blocks, n_col_blocks),
            in_specs=[
                pl.BlockSpec(
                    (_ROWS_PER_BLOCK, _COLS_PER_BLOCK), lambda r, c, off: (r, c)
                ),
            ],
            out_specs=[
                pl.BlockSpec((_ROWS_PER_BLOCK, 1), lambda r, c, off: (r, 0)),
                pl.BlockSpec((_ROWS_PER_BLOCK, 1), lambda r, c, off: (r, 0)),
            ],
            scratch_shapes=[
                pltpu.VMEM((_ROWS_PER_BLOCK, _LANES), jnp.float32),
                pltpu.VMEM((_ROWS_PER_BLOCK, _LANES), jnp.int32),
            ],
        ),
        out_shape=[
            jax.ShapeDtypeStruct((n_rows, 1), jnp.float32),
            jax.ShapeDtypeStruct((n_rows, 1), jnp.int32),
        ],
    )(jnp.asarray([offset], dtype=jnp.int32), logits)
    return outmax[:, 0], outlin[:, 0]


def kernel(logits):
    n_rows, n_cols = logits.shape
    devs = jax.devices()
    use_shards = len(devs) >= 2 and n_cols % 2 == 0

    if not use_shards:
        _, lin = _partial_sample(logits, n_cols, 0)
        return lin - _KS1

    half = n_cols // 2
    mesh = Mesh(np.asarray(devs[:2]), ("x",))
    logits = jax.lax.with_sharding_constraint(
        logits, jax.sharding.NamedSharding(mesh, P(None, "x"))
    )

    def shard_fn(lg):
        off = jax.lax.axis_index("x") * half
        m, lin = _partial_sample(lg, n_cols, off)
        return m[:, None], lin[:, None]

    m2, l2 = jax.shard_map(
        shard_fn,
        mesh=mesh,
        in_specs=P(None, "x"),
        out_specs=P(None, "x"),
        check_vma=False,
    )(logits)
    take1 = m2[:, 1] > m2[:, 0]
    lin = jnp.where(take1, l2[:, 1], l2[:, 0])
    return lin - _KS1


# speedup vs baseline: 1.8517x; 1.7907x over previous
import functools

import jax
import jax.numpy as jnp
import numpy as np
from jax.experimental import pallas as pl
from jax.experimental.pallas import tpu as pltpu
from jax.sharding import Mesh, NamedSharding, PartitionSpec as P

_ROWS_PER_BLOCK = 32
_COLS_PER_BLOCK = 16384
_LANES = 128

_TINY = np.float32(np.finfo(np.float32).tiny)
_NEG_INF = np.float32(-np.inf)
_INT_MAX = np.int32(0x7FFFFFFF)

_KS0 = 0
_KS1 = 42
_KS2 = 0x1BD11BDA ^ _KS0 ^ _KS1
_ROT0 = (13, 15, 26, 6)
_ROT1 = (17, 29, 16, 24)


def _rotl(x, r):
    return (x << jnp.uint32(r)) | (x >> jnp.uint32(32 - r))


def _threefry_bits(x1):
    x0 = x1
    x1 = _rotl(x1, _ROT0[0]) ^ x0
    for r in _ROT0[1:]:
        x0 = x0 + x1
        x1 = _rotl(x1, r) ^ x0
    x0 = x0 + jnp.uint32(_KS1)
    x1 = x1 + jnp.uint32((_KS2 + 1) & 0xFFFFFFFF)
    for r in _ROT1:
        x0 = x0 + x1
        x1 = _rotl(x1, r) ^ x0
    x0 = x0 + jnp.uint32(_KS2)
    x1 = x1 + jnp.uint32(_KS0 + 2)
    for r in _ROT0:
        x0 = x0 + x1
        x1 = _rotl(x1, r) ^ x0
    x1 = x1 + jnp.uint32(_KS1 + 3)
    for r in _ROT1:
        x0 = x0 + x1
        x1 = _rotl(x1, r) ^ x0
    x0 = x0 + jnp.uint32(_KS1)
    x1 = x1 + jnp.uint32((_KS2 + 4) & 0xFFFFFFFF)
    for r in _ROT0:
        x0 = x0 + x1
        x1 = _rotl(x1, r) ^ x0
    x0 = x0 + jnp.uint32(_KS2)
    x1 = x1 + jnp.uint32(_KS0 + 5)
    return x0 ^ x1


def _sample_kernel(
    boff_ref,
    logits_ref,
    outmax_ref,
    outlin_ref,
    max_ref,
    arg_ref,
    *,
    n_total,
    n_window_blocks,
):
    r = pl.program_id(0)
    c = pl.program_id(1)
    rows = _ROWS_PER_BLOCK

    @pl.when(c == 0)
    def _init():
        max_ref[...] = jnp.full_like(max_ref, _NEG_INF)
        arg_ref[...] = jnp.full_like(arg_ref, _INT_MAX)

    shape = (rows, _LANES)
    lane = jax.lax.broadcasted_iota(jnp.int32, shape, 1)
    row = r * rows + jax.lax.broadcasted_iota(jnp.int32, shape, 0)
    row_base = row * n_total
    col_base = (boff_ref[0] + c) * _COLS_PER_BLOCK
    linp0 = row_base + (col_base + _KS1) + lane
    bound = row_base + (n_total + _KS1)

    run_max = max_ref[...]
    run_arg = arg_ref[...]
    for j in range(_COLS_PER_BLOCK // _LANES):
        linp = linp0 + j * _LANES
        bits = _threefry_bits(linp.astype(jnp.uint32))
        fb = (bits >> jnp.uint32(9)) | jnp.uint32(0x3F800000)
        u = jax.lax.bitcast_convert_type(fb, jnp.float32) + jnp.float32(-1.0)
        u = jnp.maximum(u, _TINY)
        e = -jnp.log(u)
        val = logits_ref[:, j * _LANES : (j + 1) * _LANES] - jnp.log(e)
        val = jnp.where(linp < bound, val, _NEG_INF)
        upd = val > run_max
        run_max = jnp.where(upd, val, run_max)
        run_arg = jnp.where(upd, linp, run_arg)
    max_ref[...] = run_max
    arg_ref[...] = run_arg

    @pl.when(c == n_window_blocks - 1)
    def _finalize():
        m = max_ref[...]
        row_max = jnp.max(m, axis=1, keepdims=True)
        ids = jnp.where(m == row_max, arg_ref[...], _INT_MAX)
        outmax_ref[...] = row_max
        outlin_ref[...] = jnp.min(ids, axis=1, keepdims=True) - row_base[:, :1]


def _partial_sample(logits, block_off, n_window_blocks):
    n_rows, n_cols = logits.shape
    n_row_blocks = pl.cdiv(n_rows, _ROWS_PER_BLOCK)

    outmax, outlin = pl.pallas_call(
        functools.partial(
            _sample_kernel,
            n_total=n_cols,
            n_window_blocks=n_window_blocks,
        ),
        grid_spec=pltpu.PrefetchScalarGridSpec(
            num_scalar_prefetch=1,
            grid=(n_row_blocks, n_window_blocks),
            in_specs=[
                pl.BlockSpec(
                    (_ROWS_PER_BLOCK, _COLS_PER_BLOCK),
                    lambda r, c, boff: (r, boff[0] + c),
                ),
            ],
            out_specs=[
                pl.BlockSpec((_ROWS_PER_BLOCK, 1), lambda r, c, boff: (r, 0)),
                pl.BlockSpec((_ROWS_PER_BLOCK, 1), lambda r, c, boff: (r, 0)),
            ],
            scratch_shapes=[
                pltpu.VMEM((_ROWS_PER_BLOCK, _LANES), jnp.float32),
                pltpu.VMEM((_ROWS_PER_BLOCK, _LANES), jnp.int32),
            ],
        ),
        out_shape=[
            jax.ShapeDtypeStruct((n_rows, 1), jnp.float32),
            jax.ShapeDtypeStruct((n_rows, 1), jnp.int32),
        ],
    )(jnp.asarray([block_off], dtype=jnp.int32), logits)
    return outmax[:, 0], outlin[:, 0]


def kernel(logits):
    n_rows, n_cols = logits.shape
    n_col_blocks = pl.cdiv(n_cols, _COLS_PER_BLOCK)
    devs = jax.devices()
    use_shards = len(devs) >= 2 and n_col_blocks >= 2

    if not use_shards:
        _, lin = _partial_sample(logits, 0, n_col_blocks)
        return lin - _KS1

    blocks_per_shard = pl.cdiv(n_col_blocks, 2)
    mesh = Mesh(np.asarray(devs[:2]), ("x",))
    logits = jax.lax.with_sharding_constraint(
        logits, NamedSharding(mesh, P(None, None))
    )

    def shard_fn(lg):
        boff = jax.lax.axis_index("x") * blocks_per_shard
        m, lin = _partial_sample(lg, boff, blocks_per_shard)
        return m[:, None], lin[:, None]

    m2, l2 = jax.shard_map(
        shard_fn,
        mesh=mesh,
        in_specs=P(None, None),
        out_specs=P(None, "x"),
        check_vma=False,
    )(logits)
    take1 = m2[:, 1] > m2[:, 0]
    lin = jnp.where(take1, l2[:, 1], l2[:, 0])
    return lin - _KS1
